# Initial kernel scaffold; baseline (speedup 1.0000x reference)
import jax
import jax.numpy as jnp
from jax.experimental import pallas as pl


def kernel(x, edge_index, batch, W_rel, W_root, W_out):
    raise NotImplementedError("write your pallas kernel here")


# speedup vs baseline: 5.8616x; 5.8616x over previous
import functools

import jax
import jax.numpy as jnp
from jax import lax
from jax.experimental import pallas as pl
from jax.experimental.pallas import tpu as pltpu
from jax.experimental.pallas import tpu_sc as plsc

N = 10000
E = 320000
D = 128
H = 128
C = 10
G = 64

NC = 2
NS = 16
NW = NC * NS
EPW = E // NW
CHUNK = 80
NCHUNKS = EPW // CHUNK
NPAD = 10240
RPT = NPAD // NS
ZROWS = 128


_sc_mesh = plsc.VectorSubcoreMesh(core_axis_name="c", subcore_axis_name="s")


@functools.partial(
    pl.kernel,
    out_type=jax.ShapeDtypeStruct((NC, NPAD, D), jnp.float32),
    mesh=_sc_mesh,
    scratch_types=[
        pltpu.VMEM((CHUNK,), jnp.int32),
        pltpu.VMEM((CHUNK,), jnp.int32),
        pltpu.VMEM((CHUNK, D), jnp.float32),
        pltpu.VMEM((ZROWS, D), jnp.float32),
        pltpu.VMEM_SHARED((NPAD, D), jnp.float32),
        pltpu.SemaphoreType.DMA,
    ],
)
def _sc_agg(x_hbm, src_hbm, dst_hbm, out_hbm,
            idx_src, idx_dst, rows, zbuf, agg_sh, sem):
    cid = lax.axis_index("c")
    sid = lax.axis_index("s")
    wid = sid * NC + cid

    def _zrow(i, carry):
        def _zcol(j, c):
            zbuf[i, pl.ds(j * 16, 16)] = jnp.zeros((16,), jnp.float32)
            return c
        return lax.fori_loop(0, D // 16, _zcol, carry)
    lax.fori_loop(0, ZROWS, _zrow, 0)

    base = sid * RPT
    def _zcopy(k, carry):
        pltpu.sync_copy(zbuf, agg_sh.at[pl.ds(base + k * ZROWS, ZROWS)])
        return carry
    lax.fori_loop(0, RPT // ZROWS, _zcopy, 0)
    plsc.subcore_barrier()

    ebase = wid * EPW
    def _body(j, carry):
        off = ebase + j * CHUNK
        pltpu.sync_copy(src_hbm.at[pl.ds(off, CHUNK)], idx_src)
        pltpu.sync_copy(dst_hbm.at[pl.ds(off, CHUNK)], idx_dst)
        pltpu.async_copy(x_hbm.at[idx_src], rows, sem).wait()
        pltpu.sync_copy(rows, agg_sh.at[idx_dst], add=True)
        return carry
    lax.fori_loop(0, NCHUNKS, _body, 0)
    plsc.subcore_barrier()

    pltpu.sync_copy(agg_sh.at[pl.ds(base, RPT)],
                    out_hbm.at[cid, pl.ds(base, RPT)])


def _tc_body(aggs_ref, x_ref, batch_ref, wrel_ref, wroot_ref, wout_ref,
             out_ref):
    agg = (aggs_ref[0] + aggs_ref[1])[:N]
    h = jnp.dot(agg, wrel_ref[...], preferred_element_type=jnp.float32)
    h += jnp.dot(x_ref[...], wroot_ref[...],
                 preferred_element_type=jnp.float32)
    h = jnp.maximum(h, 0.0)
    gids = lax.broadcasted_iota(jnp.int32, (G, N), 0)
    onehot = (gids == batch_ref[...]).astype(jnp.float32)
    sums = jnp.dot(onehot, h, preferred_element_type=jnp.float32)
    counts = jnp.sum(onehot, axis=1, keepdims=True)
    pooled = sums / jnp.maximum(counts, 1.0)
    out_ref[...] = jnp.dot(pooled, wout_ref[...],
                           preferred_element_type=jnp.float32)


_tc_readout = pl.pallas_call(
    _tc_body,
    out_shape=jax.ShapeDtypeStruct((G, C), jnp.float32),
)


def kernel(x, edge_index, batch, W_rel, W_root, W_out):
    src = edge_index[0]
    dst = edge_index[1]
    agg_parts = _sc_agg(x, src, dst)
    return _tc_readout(agg_parts, x, batch.reshape(1, N),
                       W_rel, W_root, W_out)


# speedup vs baseline: 10.3129x; 1.7594x over previous
import functools

import jax
import jax.numpy as jnp
from jax import lax
from jax.experimental import pallas as pl
from jax.experimental.pallas import tpu as pltpu
from jax.experimental.pallas import tpu_sc as plsc

N = 10000
E = 320000
D = 128
H = 128
C = 10
G = 64

NC = 2
NS = 16
NW = NC * NS
EPW = E // NW
CHUNK = 80
NCHUNKS = EPW // CHUNK
NPAD = 10240
RPT = NPAD // NS
ZROWS = 32


_sc_mesh = plsc.VectorSubcoreMesh(core_axis_name="c", subcore_axis_name="s")


@functools.partial(
    pl.kernel,
    out_type=jax.ShapeDtypeStruct((NC, NPAD, D), jnp.float32),
    mesh=_sc_mesh,
    scratch_types=[
        pltpu.VMEM((NCHUNKS, CHUNK), jnp.int32),
        pltpu.VMEM((CHUNK,), jnp.int32),
        pltpu.VMEM((CHUNK,), jnp.int32),
        pltpu.VMEM((CHUNK, D), jnp.float32),
        pltpu.VMEM((CHUNK, D), jnp.float32),
        pltpu.VMEM((ZROWS, D), jnp.float32),
        pltpu.VMEM_SHARED((NPAD, D), jnp.float32),
        pltpu.SemaphoreType.DMA,
        pltpu.SemaphoreType.DMA,
        pltpu.SemaphoreType.DMA,
        pltpu.SemaphoreType.DMA,
        pltpu.SemaphoreType.DMA,
        pltpu.SemaphoreType.DMA,
    ],
)
def _sc_agg(x_hbm, src_hbm, dst_hbm, out_hbm,
            idx_src, idst0, idst1, rows0, rows1, zbuf, agg_sh,
            gs0, gs1, ss0, ss1, is0, is1):
    cid = lax.axis_index("c")
    sid = lax.axis_index("s")
    wid = sid * NC + cid

    pltpu.sync_copy(src_hbm.at[wid], idx_src)

    def _zrow(i, carry):
        def _zcol(j, c):
            zbuf[i, pl.ds(j * 16, 16)] = jnp.zeros((16,), jnp.float32)
            return c
        return lax.fori_loop(0, D // 16, _zcol, carry)
    lax.fori_loop(0, ZROWS, _zrow, 0)

    base = sid * RPT
    def _zcopy(k, carry):
        pltpu.sync_copy(zbuf, agg_sh.at[pl.ds(base + k * ZROWS, ZROWS)])
        return carry
    lax.fori_loop(0, RPT // ZROWS, _zcopy, 0)
    plsc.subcore_barrier()

    ebase = wid * EPW

    def _fire_g(j, rbuf, sem):
        pltpu.async_copy(x_hbm.at[idx_src.at[j]], rbuf, sem)

    def _wait_g(rbuf, sem):
        pltpu.make_async_copy(x_hbm.at[idx_src.at[0]], rbuf, sem).wait()

    def _fire_i(j, ibuf, sem):
        pltpu.async_copy(dst_hbm.at[pl.ds(ebase + j * CHUNK, CHUNK)],
                         ibuf, sem)

    def _wait_i(ibuf, sem):
        pltpu.make_async_copy(dst_hbm.at[pl.ds(0, CHUNK)], ibuf, sem).wait()

    def _fire_s(rbuf, ibuf, sem):
        pltpu.async_copy(rbuf, agg_sh.at[ibuf], sem, add=True)

    def _wait_s(rbuf, ibuf, sem):
        pltpu.make_async_copy(rbuf, agg_sh.at[ibuf], sem).wait()

    _fire_g(0, rows0, gs0)
    _fire_i(0, idst0, is0)
    _wait_g(rows0, gs0)
    _wait_i(idst0, is0)
    _fire_s(rows0, idst0, ss0)
    _fire_g(1, rows1, gs1)
    _fire_i(1, idst1, is1)

    def _body(jj, carry):
        j1 = 2 * jj + 1
        j2 = j1 + 1
        _wait_g(rows1, gs1)
        _wait_i(idst1, is1)
        _fire_s(rows1, idst1, ss1)
        _wait_s(rows0, idst0, ss0)
        _fire_g(j2, rows0, gs0)
        _fire_i(j2, idst0, is0)
        _wait_g(rows0, gs0)
        _wait_i(idst0, is0)
        _fire_s(rows0, idst0, ss0)
        _wait_s(rows1, idst1, ss1)
        _fire_g(j2 + 1, rows1, gs1)
        _fire_i(j2 + 1, idst1, is1)
        return carry
    lax.fori_loop(0, (NCHUNKS - 3) // 2, _body, 0)

    _wait_g(rows1, gs1)
    _wait_i(idst1, is1)
    _fire_s(rows1, idst1, ss1)
    _wait_s(rows0, idst0, ss0)
    _fire_g(NCHUNKS - 1, rows0, gs0)
    _fire_i(NCHUNKS - 1, idst0, is0)
    _wait_g(rows0, gs0)
    _wait_i(idst0, is0)
    _fire_s(rows0, idst0, ss0)
    _wait_s(rows1, idst1, ss1)
    _wait_s(rows0, idst0, ss0)
    plsc.subcore_barrier()

    pltpu.sync_copy(agg_sh.at[pl.ds(base, RPT)],
                    out_hbm.at[cid, pl.ds(base, RPT)])


def _tc_body(aggs_ref, x_ref, batch_ref, wrel_ref, wroot_ref, wout_ref,
             out_ref):
    agg = (aggs_ref[0] + aggs_ref[1])[:N]
    h = jnp.dot(agg, wrel_ref[...], preferred_element_type=jnp.float32)
    h += jnp.dot(x_ref[...], wroot_ref[...],
                 preferred_element_type=jnp.float32)
    h = jnp.maximum(h, 0.0)
    gids = lax.broadcasted_iota(jnp.int32, (G, N), 0)
    onehot = (gids == batch_ref[...]).astype(jnp.float32)
    sums = jnp.dot(onehot, h, preferred_element_type=jnp.float32)
    counts = jnp.sum(onehot, axis=1, keepdims=True)
    pooled = sums / jnp.maximum(counts, 1.0)
    out_ref[...] = jnp.dot(pooled, wout_ref[...],
                           preferred_element_type=jnp.float32)


_tc_readout = pl.pallas_call(
    _tc_body,
    out_shape=jax.ShapeDtypeStruct((G, C), jnp.float32),
)


def kernel(x, edge_index, batch, W_rel, W_root, W_out):
    src = edge_index[0].reshape(NW, NCHUNKS, CHUNK)
    dst = edge_index[1]
    agg_parts = _sc_agg(x, src, dst)
    return _tc_readout(agg_parts, x, batch.reshape(1, N),
                       W_rel, W_root, W_out)


# speedup vs baseline: 12.8172x; 1.2428x over previous
import functools

import jax
import jax.numpy as jnp
from jax import lax
from jax.experimental import pallas as pl
from jax.experimental.pallas import tpu as pltpu
from jax.experimental.pallas import tpu_sc as plsc

N = 10000
E = 320000
D = 128
H = 128
C = 10
G = 64

NC = 2
NS = 16
NW = NC * NS
EPW = E // NW
CHUNK = 80
NCHUNKS = EPW // CHUNK
NPAD = 10240
RPT = NPAD // NS
ZROWS = 32


_sc_mesh = plsc.VectorSubcoreMesh(core_axis_name="c", subcore_axis_name="s")


@functools.partial(
    pl.kernel,
    out_type=jax.ShapeDtypeStruct((NC, NPAD, D), jnp.float32),
    mesh=_sc_mesh,
    scratch_types=[
        pltpu.VMEM((2, CHUNK), jnp.int32),
        pltpu.VMEM((2, CHUNK), jnp.int32),
        pltpu.VMEM((2, CHUNK), jnp.int32),
        pltpu.VMEM((2, CHUNK), jnp.int32),
        pltpu.VMEM((CHUNK, D), jnp.float32),
        pltpu.VMEM((CHUNK, D), jnp.float32),
        pltpu.VMEM((CHUNK, D), jnp.float32),
        pltpu.VMEM((CHUNK, D), jnp.float32),
        pltpu.VMEM((ZROWS, D), jnp.float32),
        pltpu.VMEM_SHARED((NPAD, D), jnp.float32),
        pltpu.SemaphoreType.DMA,
        pltpu.SemaphoreType.DMA,
        pltpu.SemaphoreType.DMA,
        pltpu.SemaphoreType.DMA,
        pltpu.SemaphoreType.DMA,
        pltpu.SemaphoreType.DMA,
        pltpu.SemaphoreType.DMA,
        pltpu.SemaphoreType.DMA,
        pltpu.SemaphoreType.DMA,
        pltpu.SemaphoreType.DMA,
        pltpu.SemaphoreType.DMA,
        pltpu.SemaphoreType.DMA,
    ],
)
def _sc_agg(x_hbm, eidx_hbm, out_hbm,
            ib0, ib1, ib2, ib3, rb0, rb1, rb2, rb3, zbuf, agg_sh,
            gs0, gs1, gs2, gs3, ss0, ss1, ss2, ss3, is0, is1, is2, is3):
    cid = lax.axis_index("c")
    sid = lax.axis_index("s")
    wid = sid * NC + cid

    IB = (ib0, ib1, ib2, ib3)
    RB = (rb0, rb1, rb2, rb3)
    GS = (gs0, gs1, gs2, gs3)
    SS = (ss0, ss1, ss2, ss3)
    IS = (is0, is1, is2, is3)

    def _zrow(i, carry):
        def _zcol(j, c):
            zbuf[i, pl.ds(j * 16, 16)] = jnp.zeros((16,), jnp.float32)
            return c
        return lax.fori_loop(0, D // 16, _zcol, carry)
    lax.fori_loop(0, ZROWS, _zrow, 0)

    base = sid * RPT
    def _zcopy(k, carry):
        pltpu.sync_copy(zbuf, agg_sh.at[pl.ds(base + k * ZROWS, ZROWS)])
        return carry
    lax.fori_loop(0, RPT // ZROWS, _zcopy, 0)
    plsc.subcore_barrier()

    def _fire_i(j, b):
        pltpu.async_copy(eidx_hbm.at[wid, j], IB[b], IS[b])

    def _wait_i(b):
        pltpu.make_async_copy(eidx_hbm.at[0, 0], IB[b], IS[b]).wait()

    def _fire_g(b):
        pltpu.async_copy(x_hbm.at[IB[b].at[0]], RB[b], GS[b])

    def _wait_g(b):
        pltpu.make_async_copy(x_hbm.at[IB[b].at[0]], RB[b], GS[b]).wait()

    def _fire_s(b):
        pltpu.async_copy(RB[b], agg_sh.at[IB[b].at[1]], SS[b], add=True)

    def _wait_s(b):
        pltpu.make_async_copy(RB[b], agg_sh.at[IB[b].at[1]], SS[b]).wait()

    def _step(j, b, prefetch_j):
        _wait_g(b)
        _fire_s(b)
        if prefetch_j is not None:
            b2 = (b + 2) % 4
            _wait_s(b2)
            _fire_i(prefetch_j, b2)
            _wait_i(b2)
            _fire_g(b2)

    for b in (0, 1):
        _fire_i(b, b)
        _wait_i(b)
        _fire_g(b)
    for j in (0, 1):
        _wait_g(j)
        _fire_s(j)
        b2 = j + 2
        _fire_i(j + 2, b2)
        _wait_i(b2)
        _fire_g(b2)

    def _body(jj, carry):
        j0 = 4 * jj + 2
        _step(j0, 2, j0 + 2)
        _step(j0 + 1, 3, j0 + 3)
        _step(j0 + 2, 0, j0 + 4)
        _step(j0 + 3, 1, j0 + 5)
        return carry
    lax.fori_loop(0, (NCHUNKS - 5) // 4, _body, 0)

    _step(NCHUNKS - 3, 2, NCHUNKS - 1)
    _step(NCHUNKS - 2, 3, None)
    _step(NCHUNKS - 1, 0, None)
    _wait_s(1)
    _wait_s(2)
    _wait_s(3)
    _wait_s(0)
    plsc.subcore_barrier()

    pltpu.sync_copy(agg_sh.at[pl.ds(base, RPT)],
                    out_hbm.at[cid, pl.ds(base, RPT)])


def _tc_body(aggs_ref, x_ref, batch_ref, wrel_ref, wroot_ref, wout_ref,
             out_ref):
    agg = (aggs_ref[0] + aggs_ref[1])[:N]
    h = jnp.dot(agg, wrel_ref[...], preferred_element_type=jnp.float32)
    h += jnp.dot(x_ref[...], wroot_ref[...],
                 preferred_element_type=jnp.float32)
    h = jnp.maximum(h, 0.0)
    gids = lax.broadcasted_iota(jnp.int32, (G, N), 0)
    onehot = (gids == batch_ref[...]).astype(jnp.float32)
    sums = jnp.dot(onehot, h, preferred_element_type=jnp.float32)
    counts = jnp.sum(onehot, axis=1, keepdims=True)
    pooled = sums / jnp.maximum(counts, 1.0)
    out_ref[...] = jnp.dot(pooled, wout_ref[...],
                           preferred_element_type=jnp.float32)


_tc_readout = pl.pallas_call(
    _tc_body,
    out_shape=jax.ShapeDtypeStruct((G, C), jnp.float32),
)


def kernel(x, edge_index, batch, W_rel, W_root, W_out):
    eidx = edge_index.reshape(2, NW, NCHUNKS, CHUNK).transpose(1, 2, 0, 3)
    agg_parts = _sc_agg(x, eidx)
    return _tc_readout(agg_parts, x, batch.reshape(1, N),
                       W_rel, W_root, W_out)


# speedup vs baseline: 13.7195x; 1.0704x over previous
import functools

import jax
import jax.numpy as jnp
from jax import lax
from jax.experimental import pallas as pl
from jax.experimental.pallas import tpu as pltpu
from jax.experimental.pallas import tpu_sc as plsc

N = 10000
E = 320000
D = 128
H = 128
C = 10
G = 64

NC = 2
NS = 16
NW = NC * NS
EPW = E // NW
CHUNK = 80
NCHUNKS = EPW // CHUNK
NPAD = 10240
RPT = NPAD // NS
ZROWS = 32


_sc_mesh = plsc.VectorSubcoreMesh(core_axis_name="c", subcore_axis_name="s")


@functools.partial(
    pl.kernel,
    out_type=jax.ShapeDtypeStruct((NC, NPAD, D), jnp.float32),
    mesh=_sc_mesh,
    scratch_types=(
        [pltpu.VMEM((2, CHUNK), jnp.int32)] * 8
        + [pltpu.VMEM((CHUNK, D), jnp.float32)] * 4
        + [
            pltpu.VMEM((ZROWS, D), jnp.float32),
            pltpu.VMEM_SHARED((NPAD, D), jnp.float32),
        ]
        + [pltpu.SemaphoreType.DMA] * 16
    ),
)
def _sc_agg(x_hbm, eidx_hbm, out_hbm,
            ib0, ib1, ib2, ib3, ib4, ib5, ib6, ib7,
            rb0, rb1, rb2, rb3, zbuf, agg_sh,
            gs0, gs1, gs2, gs3, ss0, ss1, ss2, ss3,
            is0, is1, is2, is3, is4, is5, is6, is7):
    cid = lax.axis_index("c")
    sid = lax.axis_index("s")
    wid = sid * NC + cid

    IB = (ib0, ib1, ib2, ib3, ib4, ib5, ib6, ib7)
    RB = (rb0, rb1, rb2, rb3)
    GS = (gs0, gs1, gs2, gs3)
    SS = (ss0, ss1, ss2, ss3)
    IS = (is0, is1, is2, is3, is4, is5, is6, is7)

    def _zrow(i, carry):
        def _zcol(j, c):
            zbuf[i, pl.ds(j * 16, 16)] = jnp.zeros((16,), jnp.float32)
            return c
        return lax.fori_loop(0, D // 16, _zcol, carry)
    lax.fori_loop(0, ZROWS, _zrow, 0)

    base = sid * RPT
    def _zcopy(k, carry):
        pltpu.sync_copy(zbuf, agg_sh.at[pl.ds(base + k * ZROWS, ZROWS)])
        return carry
    lax.fori_loop(0, RPT // ZROWS, _zcopy, 0)
    plsc.subcore_barrier()

    def _fire_i(j, s):
        pltpu.async_copy(eidx_hbm.at[wid, j], IB[s], IS[s])

    def _wait_i(s):
        pltpu.make_async_copy(eidx_hbm.at[0, 0], IB[s], IS[s]).wait()

    def _fire_g(islot, b):
        pltpu.async_copy(x_hbm.at[IB[islot].at[0]], RB[b], GS[b])

    def _wait_g(b):
        pltpu.make_async_copy(x_hbm.at[IB[0].at[0]], RB[b], GS[b]).wait()

    def _fire_s(islot, b):
        pltpu.async_copy(RB[b], agg_sh.at[IB[islot].at[1]], SS[b],
                         add=True)

    def _wait_s(b):
        pltpu.make_async_copy(RB[b], agg_sh.at[IB[0].at[1]], SS[b]).wait()

    def _generic_step(j, b, islot, pf_j, pf_slot, g_j_slot, drain=True):
        _wait_g(b)
        _fire_s(islot, b)
        if drain:
            _wait_s((b + 2) % 4)
        if pf_j is not None:
            _fire_i(pf_j, pf_slot)
        if g_j_slot is not None:
            _wait_i(g_j_slot)
            _fire_g(g_j_slot, (b + 2) % 4)

    for s in range(6):
        _fire_i(s, s)
    for b in (0, 1):
        _wait_i(b)
        _fire_g(b, b)
    _generic_step(0, 0, 0, 6, 6, 2, drain=False)
    _generic_step(1, 1, 1, 7, 7, 3, drain=False)

    def _body(jj, carry):
        j0 = 8 * jj + 2
        for k in range(8):
            j = j0 + k
            b = (2 + k) % 4
            islot = (2 + k) % 8
            _generic_step(j, b, islot, j + 6, (islot + 6) % 8,
                          (islot + 2) % 8)
        return carry
    lax.fori_loop(0, (NCHUNKS - 11) // 8, _body, 0)

    for j in range(NCHUNKS - 11, NCHUNKS):
        b = j % 4
        islot = j % 8
        pf = j + 6 if j + 6 < NCHUNKS else None
        gslot = (islot + 2) % 8 if j + 2 < NCHUNKS else None
        _generic_step(j, b, islot, pf, (islot + 6) % 8, gslot)
    _wait_s((NCHUNKS - 2) % 4)
    _wait_s((NCHUNKS - 1) % 4)
    plsc.subcore_barrier()

    pltpu.sync_copy(agg_sh.at[pl.ds(base, RPT)],
                    out_hbm.at[cid, pl.ds(base, RPT)])


def _tc_body(aggs_ref, x_ref, batch_ref, wrel_ref, wroot_ref, wout_ref,
             out_ref):
    agg = (aggs_ref[0] + aggs_ref[1])[:N]
    h = jnp.dot(agg, wrel_ref[...], preferred_element_type=jnp.float32)
    h += jnp.dot(x_ref[...], wroot_ref[...],
                 preferred_element_type=jnp.float32)
    h = jnp.maximum(h, 0.0)
    gids = lax.broadcasted_iota(jnp.int32, (G, N), 0)
    onehot = (gids == batch_ref[...]).astype(jnp.float32)
    sums = jnp.dot(onehot, h, preferred_element_type=jnp.float32)
    counts = jnp.sum(onehot, axis=1, keepdims=True)
    pooled = sums / jnp.maximum(counts, 1.0)
    out_ref[...] = jnp.dot(pooled, wout_ref[...],
                           preferred_element_type=jnp.float32)


_tc_readout = pl.pallas_call(
    _tc_body,
    out_shape=jax.ShapeDtypeStruct((G, C), jnp.float32),
)


def kernel(x, edge_index, batch, W_rel, W_root, W_out):
    eidx = edge_index.reshape(2, NW, NCHUNKS, CHUNK).transpose(1, 2, 0, 3)
    agg_parts = _sc_agg(x, eidx)
    return _tc_readout(agg_parts, x, batch.reshape(1, N),
                       W_rel, W_root, W_out)
